# baseline (speedup 1.0000x reference)
import functools
import math

import jax
import jax.numpy as jnp
from jax import lax
from jax.experimental import pallas as pl
from jax.experimental.pallas import tpu as pltpu
from jax.experimental.pallas import tpu_sc as plsc

D_MODEL = 64
SCALE = math.sqrt(D_MODEL)

NC = 2
NS = 16
NW = NC * NS

CHUNK = 128
NBUF = 4


@functools.lru_cache(maxsize=None)
def _build(T, NBLK, D):
    assert NBLK == NW and D == D_MODEL and T % NBUF == 0
    n_groups = T // NBUF
    assert n_groups >= 2
    DT = D // 8

    mesh = plsc.VectorSubcoreMesh(core_axis_name="c", subcore_axis_name="s")

    @functools.partial(
        pl.kernel,
        mesh=mesh,
        out_type=jax.ShapeDtypeStruct((T, DT, NBLK, 8, CHUNK), jnp.float32),
        compiler_params=pltpu.CompilerParams(
            use_tc_tiling_on_sc=False, needs_layout_passes=False),
        scratch_types=(
            [pltpu.VMEM((T, CHUNK), jnp.int32)]
            + [pltpu.VMEM((CHUNK, D + 8), jnp.float32) for _ in range(NBUF)]
            + [pltpu.VMEM((1, DT, 1, 8, CHUNK + 1), jnp.float32)
               for _ in range(NBUF)]
            + [pltpu.SemaphoreType.DMA for _ in range(2 * NBUF)]
        ),
    )
    def embed(lut_hbm, idxt_hbm, out_hbm,
              idx_v,
              g0, g1, g2, g3, o0, o1, o2, o3,
              gs0, gs1, gs2, gs3, os0, os1, os2, os3):
        gbuf = (g0, g1, g2, g3)
        obuf = (o0, o1, o2, o3)
        gsem = (gs0, gs1, gs2, gs3)
        osem = (os0, os1, os2, os3)

        wid = lax.axis_index("s") * NC + lax.axis_index("c")

        pltpu.sync_copy(idxt_hbm.at[:, pl.ds(wid * CHUNK, CHUNK)], idx_v)

        lane = lax.iota(jnp.int32, 16)
        zero = jnp.zeros((16,), dtype=jnp.int32)
        a_idx = [(lane + 16 * k) // 8 for k in range(D // 16)]
        b_idx = [(lane + 16 * k) % 8 for k in range(D // 16)]

        def start_gather(t, b):
            pltpu.async_copy(lut_hbm.at[idx_v.at[t]], gbuf[b], gsem[b])

        def wait_gather(b):
            pltpu.make_async_copy(lut_hbm.at[idx_v.at[0]], gbuf[b],
                                  gsem[b]).wait()

        def start_out(t, b):
            pltpu.async_copy(
                obuf[b].at[:, :, :, :, pl.ds(0, CHUNK)],
                out_hbm.at[pl.ds(t, 1), :, pl.ds(wid, 1), :, :],
                osem[b])

        def wait_out(b):
            pltpu.make_async_copy(
                obuf[b].at[:, :, :, :, pl.ds(0, CHUNK)],
                out_hbm.at[pl.ds(0, 1), :, pl.ds(0, 1), :, :],
                osem[b]).wait()

        def transpose_scale(b):
            src = gbuf[b]
            dst = obuf[b]

            @plsc.parallel_loop(0, CHUNK, 1, unroll=4)
            def body(j):
                col = jnp.full((16,), j, dtype=jnp.int32)
                for k in range(D // 16):
                    v = src[j, pl.ds(16 * k, 16)]
                    plsc.store_scatter(dst, [zero, a_idx[k], zero,
                                             b_idx[k], col], v * SCALE)

        for b in range(NBUF):
            start_gather(b, b)

        for b in range(NBUF):
            wait_gather(b)
            transpose_scale(b)
            start_out(b, b)
            start_gather(b + NBUF, b)

        def group(g, carry):
            for b in range(NBUF):
                t = g * NBUF + b
                wait_gather(b)
                wait_out(b)
                transpose_scale(b)
                start_out(t, b)
                start_gather(t + NBUF, b)
            return carry

        lax.fori_loop(1, n_groups - 1, group, 0)

        for b in range(NBUF):
            t = (n_groups - 1) * NBUF + b
            wait_gather(b)
            wait_out(b)
            transpose_scale(b)
            start_out(t, b)

        for b in range(NBUF):
            wait_out(b)

    return embed


def kernel(input, lut):
    B, T = input.shape
    D = lut.shape[1]
    idxt = jnp.transpose(input).astype(jnp.int32)
    lut_p = jnp.pad(lut, ((0, 0), (0, 8)))
    phys = _build(T, B // CHUNK, D)(lut_p, idxt)
    out = jnp.transpose(phys, (2, 4, 0, 1, 3))
    return out.reshape(B, T, D)


# speedup vs baseline: 1.6307x; 1.6307x over previous
import functools
import math

import jax
import jax.numpy as jnp
from jax import lax
from jax.experimental import pallas as pl
from jax.experimental.pallas import tpu as pltpu
from jax.experimental.pallas import tpu_sc as plsc

D_MODEL = 64
SCALE = math.sqrt(D_MODEL)

NC = 2
NS = 16
NW = NC * NS

CHUNK = 128
NBUF = 4


@functools.lru_cache(maxsize=None)
def _build(T, NBLK, D):
    assert NBLK == NW and D == D_MODEL and T % NBUF == 0
    n_groups = T // NBUF
    assert n_groups >= 2
    DT = D // 8

    mesh = plsc.VectorSubcoreMesh(core_axis_name="c", subcore_axis_name="s")

    @functools.partial(
        pl.kernel,
        mesh=mesh,
        out_type=jax.ShapeDtypeStruct((T, DT, NBLK, 8, CHUNK), jnp.float32),
        compiler_params=pltpu.CompilerParams(
            use_tc_tiling_on_sc=False, needs_layout_passes=False),
        scratch_types=(
            [pltpu.VMEM((T, CHUNK), jnp.int32)]
            + [pltpu.VMEM((CHUNK, 2 * D), jnp.float32) for _ in range(NBUF)]
            + [pltpu.VMEM((1, DT, 1, 8, CHUNK + 1), jnp.float32)
               for _ in range(NBUF)]
            + [pltpu.SemaphoreType.DMA for _ in range(2 * NBUF)]
        ),
    )
    def embed(lut_hbm, idxt_hbm, out_hbm,
              idx_v,
              g0, g1, g2, g3, o0, o1, o2, o3,
              gs0, gs1, gs2, gs3, os0, os1, os2, os3):
        gbuf = (g0, g1, g2, g3)
        obuf = (o0, o1, o2, o3)
        gsem = (gs0, gs1, gs2, gs3)
        osem = (os0, os1, os2, os3)

        wid = lax.axis_index("s") * NC + lax.axis_index("c")

        pltpu.sync_copy(idxt_hbm.at[:, pl.ds(wid * CHUNK, CHUNK)], idx_v)

        lane = lax.iota(jnp.int32, 16)
        zero = jnp.zeros((16,), dtype=jnp.int32)
        a_idx = [(lane + 16 * k) // 8 for k in range(D // 16)]
        b_idx = [(lane + 16 * k) % 8 for k in range(D // 16)]

        def start_gather(t, b):
            pltpu.async_copy(lut_hbm.at[idx_v.at[t]], gbuf[b], gsem[b])

        def wait_gather(b):
            pltpu.make_async_copy(lut_hbm.at[idx_v.at[0]], gbuf[b],
                                  gsem[b]).wait()

        def start_out(t, b):
            pltpu.async_copy(
                obuf[b].at[:, :, :, :, pl.ds(0, CHUNK)],
                out_hbm.at[pl.ds(t, 1), :, pl.ds(wid, 1), :, :],
                osem[b])

        def wait_out(b):
            pltpu.make_async_copy(
                obuf[b].at[:, :, :, :, pl.ds(0, CHUNK)],
                out_hbm.at[pl.ds(0, 1), :, pl.ds(0, 1), :, :],
                osem[b]).wait()

        def transpose_scale(b):
            src = gbuf[b]
            dst = obuf[b]

            @plsc.parallel_loop(0, CHUNK, 1, unroll=4)
            def body(j):
                col = jnp.full((16,), j, dtype=jnp.int32)
                for k in range(D // 16):
                    v = src[j, pl.ds(16 * k, 16)]
                    plsc.store_scatter(dst, [zero, a_idx[k], zero,
                                             b_idx[k], col], v * SCALE)

        for b in range(NBUF):
            start_gather(b, b)

        for b in range(NBUF):
            wait_gather(b)
            transpose_scale(b)
            start_out(b, b)
            start_gather(b + NBUF, b)

        def group(g, carry):
            for b in range(NBUF):
                t = g * NBUF + b
                wait_gather(b)
                wait_out(b)
                transpose_scale(b)
                start_out(t, b)
                start_gather(t + NBUF, b)
            return carry

        lax.fori_loop(1, n_groups - 1, group, 0)

        for b in range(NBUF):
            t = (n_groups - 1) * NBUF + b
            wait_gather(b)
            wait_out(b)
            transpose_scale(b)
            start_out(t, b)

        for b in range(NBUF):
            wait_out(b)

    return embed


def kernel(input, lut):
    B, T = input.shape
    D = lut.shape[1]
    idxt = jnp.transpose(input).astype(jnp.int32)
    lut_p = jnp.pad(lut, ((0, 0), (0, D)))
    phys = _build(T, B // CHUNK, D)(lut_p, idxt)
    out = jnp.transpose(phys, (2, 4, 0, 1, 3))
    return out.reshape(B, T, D)
